# Initial kernel scaffold; baseline (speedup 1.0000x reference)
import jax
import jax.numpy as jnp
from jax.experimental import pallas as pl


def kernel(inputs, embeddings):
    raise NotImplementedError("write your pallas kernel here")


# speedup vs baseline: 4.5881x; 4.5881x over previous
import functools

import jax
import jax.numpy as jnp
from jax import lax
from jax.experimental import pallas as pl
from jax.experimental.pallas import tpu as pltpu
from jax.experimental.pallas import tpu_sc as plsc

NB = 512


def _argmin_body(x_ref, w_ref, idx_ref, *, K):
    x = x_ref[0]
    w = w_ref[0]
    xsq = jnp.sum(x * x, axis=1, keepdims=True)
    wsq = jnp.sum(w * w, axis=0, keepdims=True)
    scores = (
        xsq
        - 2.0 * jnp.dot(x, w, preferred_element_type=jnp.float32)
        + wsq
    )
    m = jnp.min(scores, axis=1, keepdims=True)
    ii = lax.broadcasted_iota(jnp.int32, scores.shape, 1)
    idx = jnp.min(jnp.where(scores == m, ii, K), axis=1)
    v = pl.program_id(0)
    idx_ref[0, 0, :] = idx + v * K


def _compute_indices(inputs, embeddings):
    V, N, D = inputs.shape
    K = embeddings.shape[2]
    nblks = N // NB
    idx3 = pl.pallas_call(
        functools.partial(_argmin_body, K=K),
        grid=(V, nblks),
        in_specs=[
            pl.BlockSpec((1, NB, D), lambda v, nb: (v, nb, 0)),
            pl.BlockSpec((1, D, K), lambda v, nb: (v, 0, 0)),
        ],
        out_specs=pl.BlockSpec((1, 1, NB), lambda v, nb: (v * (N // NB) + nb, 0, 0)),
        out_shape=jax.ShapeDtypeStruct((V * nblks, 1, NB), jnp.int32),
    )(inputs, embeddings)
    return idx3.reshape(V * N)


def _make_sc_gather(B, D):
    info = plsc.get_sparse_core_info()
    nw = info.num_cores * info.num_subcores
    b_per_w = B // nw
    chunk = 128
    nchunks = b_per_w // chunk
    mesh = plsc.VectorSubcoreMesh(core_axis_name="c", subcore_axis_name="s")

    @functools.partial(
        pl.kernel,
        mesh=mesh,
        compiler_params=pltpu.CompilerParams(use_tc_tiling_on_sc=False),
        out_type=jax.ShapeDtypeStruct((B, D), jnp.float32),
        scratch_types=[
            pltpu.VMEM((chunk,), jnp.int32),
            pltpu.VMEM((chunk, D), jnp.float32),
            pltpu.SemaphoreType.DMA,
        ],
    )
    def gather(table_hbm, idx_hbm, out_hbm, idx_v, rows_v, sem):
        wid = lax.axis_index("s") * info.num_cores + lax.axis_index("c")
        base = wid * b_per_w

        def body(i, carry):
            off = base + i * chunk
            pltpu.sync_copy(idx_hbm.at[pl.ds(off, chunk)], idx_v)
            pltpu.async_copy(table_hbm.at[idx_v], rows_v, sem).wait()
            pltpu.sync_copy(rows_v, out_hbm.at[pl.ds(off, chunk)])
            return carry

        lax.fori_loop(0, nchunks, body, 0)

    return gather


def kernel(inputs, embeddings):
    V, N, D = inputs.shape
    K = embeddings.shape[2]
    idx = _compute_indices(inputs, embeddings)
    table = jnp.transpose(embeddings, (0, 2, 1)).reshape(V * K, D)
    out = _make_sc_gather(V * N, D)(table, idx)
    return out.reshape(V, N, D)
